# baseline (speedup 1.0000x reference)
import functools

import jax
import jax.numpy as jnp
from jax import lax
from jax.experimental import pallas as pl
from jax.experimental.pallas import tpu as pltpu
from jax.experimental.pallas import tpu_sc as plsc

VOCAB = 100000
N_POS = 4096
HIDDEN = 1024
B, L = 4, 4096

NC, NS, LANES = 2, 16, 16
NW = NC * NS
PPW = L // NW
CP = 16
NPC = PPW // CP
NB = B


def _embed():
    mesh = plsc.VectorSubcoreMesh(core_axis_name="c", subcore_axis_name="s")

    @functools.partial(
        pl.kernel,
        mesh=mesh,
        out_type=jax.ShapeDtypeStruct((B * L, HIDDEN), jnp.float32),
        scratch_types=[
            pltpu.VMEM((NB, PPW), jnp.int32),
            pltpu.VMEM((CP, HIDDEN), jnp.float32),
            pltpu.VMEM((CP, HIDDEN), jnp.float32),
            pltpu.VMEM((CP, HIDDEN), jnp.float32),
            pltpu.VMEM((CP, HIDDEN), jnp.float32),
            pltpu.VMEM((CP, HIDDEN), jnp.float32),
            pltpu.SemaphoreType.DMA,
            pltpu.SemaphoreType.DMA,
            pltpu.SemaphoreType.DMA,
            pltpu.SemaphoreType.DMA,
            pltpu.SemaphoreType.DMA,
            pltpu.SemaphoreType.DMA,
            pltpu.SemaphoreType.DMA,
            pltpu.SemaphoreType.DMA,
        ],
    )
    def body(ids_hbm, wte_hbm, wpe_hbm, out_hbm, idx_v,
             g0, g1, g2, g3, pbuf, sg0, sg1, sg2, sg3, ss0, ss1, ss2, ss3):
        G = (g0, g1, g2, g3)
        SG = (sg0, sg1, sg2, sg3)
        SS = (ss0, ss1, ss2, ss3)
        wid = lax.axis_index("s") * NC + lax.axis_index("c")
        pos_base = wid * PPW
        for b in range(NB):
            pltpu.sync_copy(ids_hbm.at[b, pl.ds(pos_base, PPW)], idx_v.at[b])

        def gather(pc, b):
            return pltpu.make_async_copy(
                wte_hbm.at[idx_v.at[b, pl.ds(pc * CP, CP)]], G[b], SG[b])

        def store(pc, b):
            off = b * L + pos_base + pc * CP
            return pltpu.make_async_copy(G[b], out_hbm.at[pl.ds(off, CP)], SS[b])

        def add_pbuf(b):
            gb = G[b]

            def row(r, c):
                for j in range(HIDDEN // LANES):
                    sl = pl.ds(j * LANES, LANES)
                    gb[r, sl] = gb[r, sl] + pbuf[r, sl]
                return c

            lax.fori_loop(0, CP, row, 0)

        def load_pbuf(pc):
            pltpu.sync_copy(wpe_hbm.at[pl.ds(pos_base + pc * CP, CP)], pbuf)

        def step(pc, b, wait_prev_store, prefetch):
            if b < NB - 1:
                if wait_prev_store:
                    store(pc - 1, b + 1).wait()
                gather(pc, b + 1).start()
            elif prefetch:
                store(pc, 0).wait()
                gather(pc + 1, 0).start()
            gather(pc, b).wait()
            add_pbuf(b)
            store(pc, b).start()

        gather(0, 0).start()
        load_pbuf(0)
        for b in range(NB):
            step(0, b, wait_prev_store=False, prefetch=True)

        def group(pc, carry):
            load_pbuf(pc)
            for b in range(NB):
                step(pc, b, wait_prev_store=True, prefetch=True)
            return carry

        lax.fori_loop(1, NPC - 1, group, 0)

        load_pbuf(NPC - 1)
        for b in range(NB):
            step(NPC - 1, b, wait_prev_store=True, prefetch=False)
        for b in range(NB):
            store(NPC - 1, b).wait()

    return body


def kernel(input_ids, wte, wpe):
    out = _embed()(input_ids.astype(jnp.int32), wte, wpe)
    return out.reshape(B, L, HIDDEN)
